# baseline (speedup 1.0000x reference)
import functools

import jax
import jax.numpy as jnp
from jax import lax
from jax.experimental import pallas as pl
from jax.experimental.pallas import tpu as pltpu
from jax.experimental.pallas import tpu_sc as plsc

C1 = 384
C2 = 384
C = 192
E = 4
TOPK = 2
H = 56
W = 56
HP = H + 2
WP = W + 2
NP = HP * WP
NI = (H - 1) * WP + W
OFF0 = WP + 1
STARTS = tuple(i * WP + j for i in range(3) for j in range(3))
L = 16


def _silu(v):
    return v * jax.nn.sigmoid(v)


def _cv1_body(x_ref, W1_ref, b1_ref, t_ref, pool_ref):
    xb = x_ref[0].astype(jnp.bfloat16)
    t = jnp.dot(W1_ref[...], xb, preferred_element_type=jnp.float32) + b1_ref[...]
    t = _silu(t)
    t_ref[0] = t.astype(jnp.bfloat16)
    pool_ref[0] = jnp.sum(t[C:], axis=1, keepdims=True) * jnp.float32(1.0 / (H * W))


def _router_sc_body(pool_hbm, wr_hbm, br_hbm, idx_hbm, wts_hbm,
                    pool_v, wr_v, br_v, idx_v, wts_v):
    cid = lax.axis_index("c")
    sid = lax.axis_index("s")

    @pl.when((cid == 0) & (sid == 0))
    def _():
        pltpu.sync_copy(pool_hbm, pool_v)
        pltpu.sync_copy(wr_hbm, wr_v)
        pltpu.sync_copy(br_hbm, br_v)
        lanes = lax.iota(jnp.int32, L)
        br_vec = br_v[...]
        idx_acc = jnp.zeros((L,), jnp.int32)
        num_acc = jnp.zeros((L,), jnp.float32)
        den_acc = jnp.ones((L,), jnp.float32)
        for b in range(4):
            lv = jnp.zeros((L,), jnp.float32)
            for e in range(E):
                acc = jnp.zeros((L,), jnp.float32)
                for k in range(C // L):
                    acc = acc + (pool_v[pl.ds(b * C + k * L, L)]
                                 * wr_v[pl.ds(e * C + k * L, L)])
                le = jnp.sum(acc)
                lv = jnp.where(lanes == e, le, lv)
            lv = lv + br_vec
            lvm = jnp.where(lanes < E, lv, jnp.float32(-1e30))
            m = jnp.max(lvm)
            ex = jnp.exp(lvm - m)
            p1 = jnp.max(ex)
            i1 = jnp.min(jnp.where(ex >= p1, lanes, E))
            pm = jnp.where(lanes == i1, jnp.float32(-1.0), ex)
            p2 = jnp.max(pm)
            i2 = jnp.min(jnp.where(pm >= p2, lanes, E))
            ssum = p1 + p2
            idx_acc = jnp.where(lanes == 2 * b, i1, idx_acc)
            idx_acc = jnp.where(lanes == 2 * b + 1, i2, idx_acc)
            num_acc = jnp.where(lanes == 2 * b, p1, num_acc)
            num_acc = jnp.where(lanes == 2 * b + 1, p2, num_acc)
            pair = (lanes == 2 * b) | (lanes == 2 * b + 1)
            den_acc = jnp.where(pair, ssum, den_acc)
        idx_v[...] = idx_acc
        wts_v[...] = num_acc / den_acc
        pltpu.sync_copy(idx_v, idx_hbm)
        pltpu.sync_copy(wts_v, wts_hbm)


def _expert_cv2_body(idx_sref, wts_sref, t_ref, Wm1_ref, Wm2_ref,
                     be1_ref, be2_ref, W2_ref, b2_ref, out_ref,
                     y1p_ref, xcat_ref, ycat_ref):
    b = pl.program_id(0)
    w1 = wts_sref[2 * b]
    w2 = wts_sref[2 * b + 1]
    ycat_ref[:2 * C] = t_ref[0]
    y1b = t_ref[0, C:]
    y1p_ref[...] = jnp.zeros((C, NP), jnp.bfloat16)
    for h in range(H):
        y1p_ref[:, (h + 1) * WP + 1:(h + 1) * WP + 1 + W] = y1b[:, h * W:(h + 1) * W]
    for sidx in range(9):
        xcat_ref[sidx * C:(sidx + 1) * C, :] = y1p_ref[:, STARTS[sidx]:STARTS[sidx] + NI]
    wcat = jnp.concatenate([Wm1_ref[0], Wm2_ref[0]], axis=0)
    acc = jnp.dot(wcat, xcat_ref[...], preferred_element_type=jnp.float32)
    e1 = _silu(acc[:C] + be1_ref[0])
    e2 = _silu(acc[C:] + be2_ref[0])
    eo = w1 * e1 + w2 * e2
    for h in range(H):
        ycat_ref[2 * C:, h * W:(h + 1) * W] = eo[:, h * WP:h * WP + W].astype(jnp.bfloat16)
    o = jnp.dot(W2_ref[...], ycat_ref[...], preferred_element_type=jnp.float32) + b2_ref[...]
    out_ref[0] = _silu(o)


def kernel(x, W1, b1, Wr, br, We, be, W2, b2):
    B = x.shape[0]
    xf = x.reshape(B, C1, H * W)
    W1r = W1.reshape(2 * C, C1).astype(jnp.bfloat16)
    b1c = b1.reshape(2 * C, 1)
    Wm2 = (We.transpose(0, 3, 4, 1, 2)
             .reshape(E, 9, C, C)
             .transpose(0, 2, 1, 3)
             .reshape(E, C, 9 * C)
             .astype(jnp.bfloat16))
    bec = be.reshape(E, C, 1)
    W2r = W2.reshape(C2, (2 + 1) * C).astype(jnp.bfloat16)
    b2c = b2.reshape(C2, 1)

    t_d, pooled = pl.pallas_call(
        _cv1_body,
        grid=(B,),
        in_specs=[
            pl.BlockSpec((1, C1, H * W), lambda b: (b, 0, 0)),
            pl.BlockSpec((2 * C, C1), lambda b: (0, 0)),
            pl.BlockSpec((2 * C, 1), lambda b: (0, 0)),
        ],
        out_specs=[
            pl.BlockSpec((1, 2 * C, H * W), lambda b: (b, 0, 0)),
            pl.BlockSpec((1, C, 1), lambda b: (b, 0, 0)),
        ],
        out_shape=[
            jax.ShapeDtypeStruct((B, 2 * C, H * W), jnp.bfloat16),
            jax.ShapeDtypeStruct((B, C, 1), jnp.float32),
        ],
        compiler_params=pltpu.CompilerParams(
            dimension_semantics=("parallel",)),
    )(xf, W1r, b1c)

    pooled_flat = pooled.reshape(B * C)
    wr_flat = Wr.reshape(E * C)
    br_pad = jnp.concatenate([br, jnp.zeros((L - E,), jnp.float32)])

    router = functools.partial(
        pl.kernel,
        mesh=plsc.VectorSubcoreMesh(core_axis_name="c", subcore_axis_name="s"),
        out_type=(
            jax.ShapeDtypeStruct((L,), jnp.int32),
            jax.ShapeDtypeStruct((L,), jnp.float32),
        ),
        scratch_types=[
            pltpu.VMEM((B * C,), jnp.float32),
            pltpu.VMEM((E * C,), jnp.float32),
            pltpu.VMEM((L,), jnp.float32),
            pltpu.VMEM((L,), jnp.int32),
            pltpu.VMEM((L,), jnp.float32),
        ],
        compiler_params=pltpu.CompilerParams(needs_layout_passes=False),
    )(_router_sc_body)
    idx16, wts16 = router(pooled_flat, wr_flat, br_pad)
    idx_flat = idx16[:B * TOPK]
    wts_flat = wts16[:B * TOPK]

    grid_spec = pltpu.PrefetchScalarGridSpec(
        num_scalar_prefetch=2,
        grid=(B,),
        in_specs=[
            pl.BlockSpec((1, 2 * C, H * W), lambda b, I, Ww: (b, 0, 0)),
            pl.BlockSpec((1, C, 9 * C), lambda b, I, Ww: (I[2 * b], 0, 0)),
            pl.BlockSpec((1, C, 9 * C), lambda b, I, Ww: (I[2 * b + 1], 0, 0)),
            pl.BlockSpec((1, C, 1), lambda b, I, Ww: (I[2 * b], 0, 0)),
            pl.BlockSpec((1, C, 1), lambda b, I, Ww: (I[2 * b + 1], 0, 0)),
            pl.BlockSpec((C2, 3 * C), lambda b, I, Ww: (0, 0)),
            pl.BlockSpec((C2, 1), lambda b, I, Ww: (0, 0)),
        ],
        out_specs=pl.BlockSpec((1, C2, H * W), lambda b, I, Ww: (b, 0, 0)),
        scratch_shapes=[
            pltpu.VMEM((C, NP), jnp.bfloat16),
            pltpu.VMEM((9 * C, NI), jnp.bfloat16),
            pltpu.VMEM((3 * C, H * W), jnp.bfloat16),
        ],
    )
    out_f = pl.pallas_call(
        _expert_cv2_body,
        grid_spec=grid_spec,
        out_shape=jax.ShapeDtypeStruct((B, C2, H * W), jnp.float32),
        compiler_params=pltpu.CompilerParams(
            dimension_semantics=("arbitrary",)),
    )(idx_flat, wts_flat, t_d, Wm2, Wm2, bec, bec, W2r, b2c)

    return out_f.reshape(B, C2, H, W)


# speedup vs baseline: 1.0311x; 1.0311x over previous
import functools

import jax
import jax.numpy as jnp
from jax import lax
from jax.experimental import pallas as pl
from jax.experimental.pallas import tpu as pltpu
from jax.experimental.pallas import tpu_sc as plsc

C1 = 384
C2 = 384
C = 192
E = 4
TOPK = 2
H = 56
W = 56
HP = H + 2
WP = W + 2
NP = HP * WP
NI = (H - 1) * WP + W
OFF0 = WP + 1
STARTS = tuple(i * WP + j for i in range(3) for j in range(3))
L = 16


def _silu(v):
    return v * jax.nn.sigmoid(v)


def _cv1_body(x_ref, W1_ref, b1_ref, y0_ref, y1p_ref, pool_ref):
    xb = x_ref[0].astype(jnp.bfloat16)
    t = jnp.dot(W1_ref[...], xb, preferred_element_type=jnp.float32) + b1_ref[...]
    t = _silu(t)
    y1 = t[C:]
    y0_ref[0] = t[:C].astype(jnp.bfloat16)
    y1b = y1.astype(jnp.bfloat16)
    y1p_ref[0] = jnp.zeros((C, NP), jnp.bfloat16)
    for h in range(H):
        y1p_ref[0, :, (h + 1) * WP + 1:(h + 1) * WP + 1 + W] = y1b[:, h * W:(h + 1) * W]
    pool_ref[0] = jnp.sum(y1, axis=1, keepdims=True) * jnp.float32(1.0 / (H * W))


def _router_sc_body(pool_hbm, wr_hbm, br_hbm, idx_hbm, wts_hbm,
                    pool_v, wr_v, br_v, idx_v, wts_v):
    cid = lax.axis_index("c")
    sid = lax.axis_index("s")

    @pl.when((cid == 0) & (sid == 0))
    def _():
        pltpu.sync_copy(pool_hbm, pool_v)
        pltpu.sync_copy(wr_hbm, wr_v)
        pltpu.sync_copy(br_hbm, br_v)
        lanes = lax.iota(jnp.int32, L)
        br_vec = br_v[...]
        idx_acc = jnp.zeros((L,), jnp.int32)
        num_acc = jnp.zeros((L,), jnp.float32)
        den_acc = jnp.ones((L,), jnp.float32)
        for b in range(4):
            lv = jnp.zeros((L,), jnp.float32)
            for e in range(E):
                acc = jnp.zeros((L,), jnp.float32)
                for k in range(C // L):
                    acc = acc + (pool_v[pl.ds(b * C + k * L, L)]
                                 * wr_v[pl.ds(e * C + k * L, L)])
                le = jnp.sum(acc)
                lv = jnp.where(lanes == e, le, lv)
            lv = lv + br_vec
            lvm = jnp.where(lanes < E, lv, jnp.float32(-1e30))
            m = jnp.max(lvm)
            ex = jnp.exp(lvm - m)
            p1 = jnp.max(ex)
            i1 = jnp.min(jnp.where(ex >= p1, lanes, E))
            pm = jnp.where(lanes == i1, jnp.float32(-1.0), ex)
            p2 = jnp.max(pm)
            i2 = jnp.min(jnp.where(pm >= p2, lanes, E))
            ssum = p1 + p2
            idx_acc = jnp.where(lanes == 2 * b, i1, idx_acc)
            idx_acc = jnp.where(lanes == 2 * b + 1, i2, idx_acc)
            num_acc = jnp.where(lanes == 2 * b, p1, num_acc)
            num_acc = jnp.where(lanes == 2 * b + 1, p2, num_acc)
            pair = (lanes == 2 * b) | (lanes == 2 * b + 1)
            den_acc = jnp.where(pair, ssum, den_acc)
        idx_v[...] = idx_acc
        wts_v[...] = num_acc / den_acc
        pltpu.sync_copy(idx_v, idx_hbm)
        pltpu.sync_copy(wts_v, wts_hbm)


def _expert_cv2_body(idx_sref, wts_sref, y0_ref, y1p_ref, Wm1_ref, Wm2_ref,
                     be1_ref, be2_ref, W2_ref, b2_ref, out_ref,
                     xcat_ref, ycat_ref):
    b = pl.program_id(0)
    w1 = wts_sref[2 * b]
    w2 = wts_sref[2 * b + 1]
    for sidx in range(9):
        xcat_ref[sidx * C:(sidx + 1) * C, :] = y1p_ref[0, :, STARTS[sidx]:STARTS[sidx] + NI]
    wcat = jnp.concatenate([Wm1_ref[0], Wm2_ref[0]], axis=0)
    acc = jnp.dot(wcat, xcat_ref[...], preferred_element_type=jnp.float32)
    e1 = _silu(acc[:C] + be1_ref[0])
    e2 = _silu(acc[C:] + be2_ref[0])
    eo = w1 * e1 + w2 * e2
    ycat_ref[:C] = y0_ref[0]
    for h in range(H):
        ycat_ref[C:2 * C, h * W:(h + 1) * W] = y1p_ref[0, :, (h + 1) * WP + 1:(h + 1) * WP + 1 + W]
        ycat_ref[2 * C:, h * W:(h + 1) * W] = eo[:, h * WP:h * WP + W].astype(jnp.bfloat16)
    o = jnp.dot(W2_ref[...], ycat_ref[...], preferred_element_type=jnp.float32) + b2_ref[...]
    out_ref[0] = _silu(o)


def kernel(x, W1, b1, Wr, br, We, be, W2, b2):
    B = x.shape[0]
    xf = x.reshape(B, C1, H * W)
    W1r = W1.reshape(2 * C, C1).astype(jnp.bfloat16)
    b1c = b1.reshape(2 * C, 1)
    Wm2 = (We.transpose(0, 3, 4, 1, 2)
             .reshape(E, 9, C, C)
             .transpose(0, 2, 1, 3)
             .reshape(E, C, 9 * C)
             .astype(jnp.bfloat16))
    bec = be.reshape(E, C, 1)
    W2r = W2.reshape(C2, (2 + 1) * C).astype(jnp.bfloat16)
    b2c = b2.reshape(C2, 1)

    y0d, y1p, pooled = pl.pallas_call(
        _cv1_body,
        grid=(B,),
        in_specs=[
            pl.BlockSpec((1, C1, H * W), lambda b: (b, 0, 0)),
            pl.BlockSpec((2 * C, C1), lambda b: (0, 0)),
            pl.BlockSpec((2 * C, 1), lambda b: (0, 0)),
        ],
        out_specs=[
            pl.BlockSpec((1, C, H * W), lambda b: (b, 0, 0)),
            pl.BlockSpec((1, C, NP), lambda b: (b, 0, 0)),
            pl.BlockSpec((1, C, 1), lambda b: (b, 0, 0)),
        ],
        out_shape=[
            jax.ShapeDtypeStruct((B, C, H * W), jnp.bfloat16),
            jax.ShapeDtypeStruct((B, C, NP), jnp.bfloat16),
            jax.ShapeDtypeStruct((B, C, 1), jnp.float32),
        ],
        compiler_params=pltpu.CompilerParams(
            dimension_semantics=("parallel",)),
    )(xf, W1r, b1c)

    pooled_flat = pooled.reshape(B * C)
    wr_flat = Wr.reshape(E * C)
    br_pad = jnp.concatenate([br, jnp.zeros((L - E,), jnp.float32)])

    router = functools.partial(
        pl.kernel,
        mesh=plsc.VectorSubcoreMesh(core_axis_name="c", subcore_axis_name="s"),
        out_type=(
            jax.ShapeDtypeStruct((L,), jnp.int32),
            jax.ShapeDtypeStruct((L,), jnp.float32),
        ),
        scratch_types=[
            pltpu.VMEM((B * C,), jnp.float32),
            pltpu.VMEM((E * C,), jnp.float32),
            pltpu.VMEM((L,), jnp.float32),
            pltpu.VMEM((L,), jnp.int32),
            pltpu.VMEM((L,), jnp.float32),
        ],
        compiler_params=pltpu.CompilerParams(needs_layout_passes=False),
    )(_router_sc_body)
    idx16, wts16 = router(pooled_flat, wr_flat, br_pad)
    idx_flat = idx16[:B * TOPK]
    wts_flat = wts16[:B * TOPK]

    grid_spec = pltpu.PrefetchScalarGridSpec(
        num_scalar_prefetch=2,
        grid=(B,),
        in_specs=[
            pl.BlockSpec((1, C, H * W), lambda b, I, Ww: (b, 0, 0)),
            pl.BlockSpec((1, C, NP), lambda b, I, Ww: (b, 0, 0)),
            pl.BlockSpec((1, C, 9 * C), lambda b, I, Ww: (I[2 * b], 0, 0)),
            pl.BlockSpec((1, C, 9 * C), lambda b, I, Ww: (I[2 * b + 1], 0, 0)),
            pl.BlockSpec((1, C, 1), lambda b, I, Ww: (I[2 * b], 0, 0)),
            pl.BlockSpec((1, C, 1), lambda b, I, Ww: (I[2 * b + 1], 0, 0)),
            pl.BlockSpec((C2, 3 * C), lambda b, I, Ww: (0, 0)),
            pl.BlockSpec((C2, 1), lambda b, I, Ww: (0, 0)),
        ],
        out_specs=pl.BlockSpec((1, C2, H * W), lambda b, I, Ww: (b, 0, 0)),
        scratch_shapes=[
            pltpu.VMEM((9 * C, NI), jnp.bfloat16),
            pltpu.VMEM((3 * C, H * W), jnp.bfloat16),
        ],
    )
    out_f = pl.pallas_call(
        _expert_cv2_body,
        grid_spec=grid_spec,
        out_shape=jax.ShapeDtypeStruct((B, C2, H * W), jnp.float32),
        compiler_params=pltpu.CompilerParams(
            dimension_semantics=("parallel",)),
    )(idx_flat, wts_flat, y0d, y1p, Wm2, Wm2, bec, bec, W2r, b2c)

    return out_f.reshape(B, C2, H, W)
